# Initial kernel scaffold; baseline (speedup 1.0000x reference)
import jax
import jax.numpy as jnp
from jax.experimental import pallas as pl


def kernel(seq, adj_row, adj_col, adj_val):
    raise NotImplementedError("write your pallas kernel here")


# speedup vs baseline: 2.1152x; 2.1152x over previous
import functools

import jax
import jax.numpy as jnp
from jax import lax
from jax.experimental import pallas as pl
from jax.experimental.pallas import tpu as pltpu
from jax.experimental.pallas import tpu_sc as plsc

N = 10000
D = 128
E = 320000

NC = 2
NS = 16
L = 16
H = D // NC
EPT = E // NS
C = 80
NCHUNK = EPT // C
RPT = N // NS


def _sc_spmm(x_lo, x_hi, adj_row, adj_col, adj_val):
    mesh = plsc.VectorSubcoreMesh(core_axis_name="c", subcore_axis_name="s")

    @functools.partial(
        pl.kernel,
        mesh=mesh,
        out_type=jax.ShapeDtypeStruct((N, D), jnp.float32),
        compiler_params=pltpu.CompilerParams(use_tc_tiling_on_sc=False),
        scratch_types=[
            pltpu.VMEM((C,), jnp.int32),
            pltpu.VMEM((C,), jnp.int32),
            pltpu.VMEM((C,), jnp.float32),
            pltpu.VMEM((C, H), jnp.float32),
            pltpu.VMEM((RPT, H), jnp.float32),
            pltpu.VMEM_SHARED((N, H), jnp.float32),
        ],
    )
    def k(xlo_hbm, xhi_hbm, row_hbm, col_hbm, val_hbm, out_hbm,
          col_v, row_v, val_v, rows_v, zbuf_v, acc_sh):
        c = lax.axis_index("c")
        s = lax.axis_index("s")

        zero16 = jnp.zeros((L,), jnp.float32)

        def zrow(i, carry):
            for g in range(H // L):
                zbuf_v[i, pl.ds(g * L, L)] = zero16
            return carry

        lax.fori_loop(0, RPT, zrow, 0)
        pltpu.sync_copy(zbuf_v, acc_sh.at[pl.ds(s * RPT, RPT)])
        plsc.subcore_barrier()

        def chunk(i, carry):
            base = s * EPT + i * C
            pltpu.sync_copy(col_hbm.at[pl.ds(base, C)], col_v)
            pltpu.sync_copy(val_hbm.at[pl.ds(base, C)], val_v)
            pltpu.sync_copy(row_hbm.at[pl.ds(base, C)], row_v)

            @pl.when(c == 0)
            def _():
                pltpu.sync_copy(xlo_hbm.at[col_v], rows_v)

            @pl.when(c == 1)
            def _():
                pltpu.sync_copy(xhi_hbm.at[col_v], rows_v)

            def scale(j, carry2):
                v16 = val_v[pl.ds(j * L, L)]
                for el in range(L):
                    v = v16[el]
                    e = j * L + el
                    for g in range(H // L):
                        sl = pl.ds(g * L, L)
                        rows_v[e, sl] = rows_v[e, sl] * v
                return carry2

            lax.fori_loop(0, C // L, scale, 0)
            pltpu.sync_copy(rows_v, acc_sh.at[row_v], add=True)
            return carry

        lax.fori_loop(0, NCHUNK, chunk, 0)
        plsc.subcore_barrier()

        lo = s * RPT
        pltpu.sync_copy(acc_sh.at[pl.ds(lo, RPT)],
                        out_hbm.at[pl.ds(lo, RPT), pl.ds(c * H, H)])

    return k(x_lo, x_hi, adj_row, adj_col, adj_val)


def kernel(seq, adj_row, adj_col, adj_val):
    x = jnp.squeeze(seq, 0)
    out = _sc_spmm(x[:, :H], x[:, H:], adj_row, adj_col, adj_val)
    return jnp.expand_dims(out, 0)


# speedup vs baseline: 4.7015x; 2.2227x over previous
import functools

import jax
import jax.numpy as jnp
from jax import lax
from jax.experimental import pallas as pl
from jax.experimental.pallas import tpu as pltpu
from jax.experimental.pallas import tpu_sc as plsc

N = 10000
D = 128
E = 320000

NC = 2
NS = 16
L = 16
H = D // NC
EPT = E // NS
C = 80
NCPT = EPT // C
RPT = N // NS


def _sc_spmm(x_lo, x_hi, row2, col2, val2):
    mesh = plsc.VectorSubcoreMesh(core_axis_name="c", subcore_axis_name="s")

    @functools.partial(
        pl.kernel,
        mesh=mesh,
        out_type=jax.ShapeDtypeStruct((N, D), jnp.float32),
        compiler_params=pltpu.CompilerParams(use_tc_tiling_on_sc=False),
        scratch_types=[
            pltpu.VMEM((NCPT, C), jnp.int32),
            pltpu.VMEM((NCPT, C), jnp.int32),
            pltpu.VMEM((NCPT, C), jnp.float32),
            pltpu.VMEM((C, H), jnp.float32),
            pltpu.VMEM((C, H), jnp.float32),
            pltpu.VMEM_SHARED((N, H), jnp.float32),
            pltpu.SemaphoreType.DMA,
            pltpu.SemaphoreType.DMA,
            pltpu.SemaphoreType.DMA,
        ],
    )
    def k(xlo_hbm, xhi_hbm, row_hbm, col_hbm, val_hbm, out_hbm,
          col_v, row_v, val_v, rows0_v, rows1_v, acc_sh,
          sem_e, sem_g0, sem_g1):
        c = lax.axis_index("c")
        s = lax.axis_index("s")
        rows_bufs = (rows0_v, rows1_v)
        sems = (sem_g0, sem_g1)

        sl_e = pl.ds(s * NCPT, NCPT)
        e_copies = [
            pltpu.make_async_copy(col_hbm.at[sl_e], col_v, sem_e),
            pltpu.make_async_copy(row_hbm.at[sl_e], row_v, sem_e),
            pltpu.make_async_copy(val_hbm.at[sl_e], val_v, sem_e),
        ]
        for cp in e_copies:
            cp.start()

        zero16 = jnp.zeros((L,), jnp.float32)

        def zrow(i, carry):
            for g in range(H // L):
                rows0_v[i, pl.ds(g * L, L)] = zero16
            return carry

        lax.fori_loop(0, C, zrow, 0)
        for q in range(RPT // C):
            pltpu.sync_copy(rows0_v, acc_sh.at[pl.ds(s * RPT + q * C, C)])
        rem = RPT % C
        if rem:
            pltpu.sync_copy(
                rows0_v.at[pl.ds(0, rem)],
                acc_sh.at[pl.ds(s * RPT + (RPT // C) * C, rem)])
        plsc.subcore_barrier()
        for cp in e_copies:
            cp.wait()

        def start_gather(i, b):
            @pl.when(c == 0)
            def _():
                pltpu.make_async_copy(
                    xlo_hbm.at[col_v.at[i]], rows_bufs[b], sems[b]).start()

            @pl.when(c == 1)
            def _():
                pltpu.make_async_copy(
                    xhi_hbm.at[col_v.at[i]], rows_bufs[b], sems[b]).start()

        def wait_gather(i, b):
            pltpu.make_async_copy(
                xlo_hbm.at[col_v.at[i]], rows_bufs[b], sems[b]).wait()

        def process(i, b):
            rows_b = rows_bufs[b]
            wait_gather(i, b)

            def scale(j, carry2):
                v16 = val_v[i, pl.ds(j * L, L)]
                for el in range(L):
                    v = v16[el]
                    e = j * L + el
                    for g in range(H // L):
                        sl = pl.ds(g * L, L)
                        rows_b[e, sl] = rows_b[e, sl] * v
                return carry2

            lax.fori_loop(0, C // L, scale, 0)
            pltpu.sync_copy(rows_b, acc_sh.at[row_v.at[i]], add=True)

        start_gather(0, 0)

        def pair(kk, carry):
            i0 = 2 * kk
            start_gather(i0 + 1, 1)
            process(i0, 0)

            @pl.when(i0 + 2 < NCPT)
            def _():
                start_gather(i0 + 2, 0)

            process(i0 + 1, 1)
            return carry

        lax.fori_loop(0, NCPT // 2, pair, 0)
        plsc.subcore_barrier()

        lo = s * RPT
        pltpu.sync_copy(acc_sh.at[pl.ds(lo, RPT)],
                        out_hbm.at[pl.ds(lo, RPT), pl.ds(c * H, H)])

    return k(x_lo, x_hi, row2, col2, val2)


def kernel(seq, adj_row, adj_col, adj_val):
    x = jnp.squeeze(seq, 0)
    out = _sc_spmm(x[:, :H], x[:, H:],
                   adj_row.reshape(-1, C), adj_col.reshape(-1, C),
                   adj_val.reshape(-1, C))
    return jnp.expand_dims(out, 0)


# speedup vs baseline: 8.8189x; 1.8758x over previous
import functools

import jax
import jax.numpy as jnp
from jax import lax
from jax.experimental import pallas as pl
from jax.experimental.pallas import tpu as pltpu
from jax.experimental.pallas import tpu_sc as plsc

N = 10000
D = 128
E = 320000

NC = 2
NS = 16
L = 16
H = D // NC
EPT = E // NS
C = 80
NCPT = EPT // C
RPT = N // NS


def _sc_spmm(x_lo, x_hi, row2, col2, val2):
    mesh = plsc.VectorSubcoreMesh(core_axis_name="c", subcore_axis_name="s")

    @functools.partial(
        pl.kernel,
        mesh=mesh,
        out_type=jax.ShapeDtypeStruct((N, D), jnp.float32),
        compiler_params=pltpu.CompilerParams(use_tc_tiling_on_sc=False),
        scratch_types=[
            pltpu.VMEM((NCPT, C), jnp.int32),
            pltpu.VMEM((NCPT, C), jnp.int32),
            pltpu.VMEM((NCPT, C), jnp.float32),
            pltpu.VMEM((C, H), jnp.float32),
            pltpu.VMEM((C, H), jnp.float32),
            pltpu.VMEM_SHARED((N, H), jnp.float32),
            pltpu.SemaphoreType.DMA,
            pltpu.SemaphoreType.DMA,
            pltpu.SemaphoreType.DMA,
        ],
    )
    def k(xlo_hbm, xhi_hbm, row_hbm, col_hbm, val_hbm, out_hbm,
          col_v, row_v, val_v, rows0_v, rows1_v, acc_sh,
          sem_e, sem_g0, sem_g1):
        c = lax.axis_index("c")
        s = lax.axis_index("s")
        rows_bufs = (rows0_v, rows1_v)
        sems = (sem_g0, sem_g1)

        sl_e = pl.ds(s * NCPT, NCPT)
        e_copies = [
            pltpu.make_async_copy(col_hbm.at[sl_e], col_v, sem_e),
            pltpu.make_async_copy(row_hbm.at[sl_e], row_v, sem_e),
            pltpu.make_async_copy(val_hbm.at[sl_e], val_v, sem_e),
        ]
        for cp in e_copies:
            cp.start()

        zero16 = jnp.zeros((L,), jnp.float32)

        def zrow(i, carry):
            for g in range(H // L):
                rows0_v[i, pl.ds(g * L, L)] = zero16
            return carry

        lax.fori_loop(0, C, zrow, 0)
        for q in range(RPT // C):
            pltpu.sync_copy(rows0_v, acc_sh.at[pl.ds(s * RPT + q * C, C)])
        rem = RPT % C
        if rem:
            pltpu.sync_copy(
                rows0_v.at[pl.ds(0, rem)],
                acc_sh.at[pl.ds(s * RPT + (RPT // C) * C, rem)])
        plsc.subcore_barrier()
        for cp in e_copies:
            cp.wait()

        def start_gather(i, b):
            @pl.when(c == 0)
            def _():
                pltpu.make_async_copy(
                    xlo_hbm.at[col_v.at[i]], rows_bufs[b], sems[b]).start()

            @pl.when(c == 1)
            def _():
                pltpu.make_async_copy(
                    xhi_hbm.at[col_v.at[i]], rows_bufs[b], sems[b]).start()

        def wait_gather(i, b):
            pltpu.make_async_copy(
                xlo_hbm.at[col_v.at[i]], rows_bufs[b], sems[b]).wait()

        def process(i, b):
            rows_b = rows_bufs[b]
            wait_gather(i, b)

            def scale(j, carry2):
                v16 = val_v[i, pl.ds(j * L, L)]
                for el in range(L):
                    v = v16[el]
                    e = j * L + el
                    loads = [rows_b[e, pl.ds(g * L, L)]
                             for g in range(H // L)]
                    prods = [x * v for x in loads]
                    for g in range(H // L):
                        rows_b[e, pl.ds(g * L, L)] = prods[g]
                return carry2

            lax.fori_loop(0, C // L, scale, 0)
            pltpu.sync_copy(rows_b, acc_sh.at[row_v.at[i]], add=True)

        start_gather(0, 0)

        def pair(kk, carry):
            i0 = 2 * kk
            start_gather(i0 + 1, 1)
            process(i0, 0)

            @pl.when(i0 + 2 < NCPT)
            def _():
                start_gather(i0 + 2, 0)

            process(i0 + 1, 1)
            return carry

        lax.fori_loop(0, NCPT // 2, pair, 0)
        plsc.subcore_barrier()

        lo = s * RPT
        pltpu.sync_copy(acc_sh.at[pl.ds(lo, RPT)],
                        out_hbm.at[pl.ds(lo, RPT), pl.ds(c * H, H)])

    return k(x_lo, x_hi, row2, col2, val2)


def kernel(seq, adj_row, adj_col, adj_val):
    x = jnp.squeeze(seq, 0)
    out = _sc_spmm(x[:, :H], x[:, H:],
                   adj_row.reshape(-1, C), adj_col.reshape(-1, C),
                   adj_val.reshape(-1, C))
    return jnp.expand_dims(out, 0)


# speedup vs baseline: 10.0231x; 1.1366x over previous
import functools

import jax
import jax.numpy as jnp
from jax import lax
from jax.experimental import pallas as pl
from jax.experimental.pallas import tpu as pltpu
from jax.experimental.pallas import tpu_sc as plsc

N = 10000
D = 128
E = 320000

NC = 2
NS = 16
L = 16
H = D // NC
EPT = E // NS
C = 80
NCPT = EPT // C
RPT = N // NS


def _sc_spmm(x_lo, x_hi, row2, col2, val2):
    mesh = plsc.VectorSubcoreMesh(core_axis_name="c", subcore_axis_name="s")

    @functools.partial(
        pl.kernel,
        mesh=mesh,
        out_type=jax.ShapeDtypeStruct((N, D), jnp.float32),
        compiler_params=pltpu.CompilerParams(use_tc_tiling_on_sc=False),
        scratch_types=[
            pltpu.VMEM((NCPT, C), jnp.int32),
            pltpu.VMEM((NCPT, C), jnp.int32),
            pltpu.VMEM((NCPT, C), jnp.float32),
            pltpu.VMEM((C, H), jnp.float32),
            pltpu.VMEM((C, H), jnp.float32),
            pltpu.VMEM_SHARED((N, H), jnp.float32),
            pltpu.SemaphoreType.DMA,
            pltpu.SemaphoreType.DMA,
            pltpu.SemaphoreType.DMA,
        ],
    )
    def k(xlo_hbm, xhi_hbm, row_hbm, col_hbm, val_hbm, out_hbm,
          col_v, row_v, val_v, rows0_v, rows1_v, acc_sh,
          sem_e, sem_g0, sem_g1):
        c = lax.axis_index("c")
        s = lax.axis_index("s")
        rows_bufs = (rows0_v, rows1_v)
        sems = (sem_g0, sem_g1)

        sl_e = pl.ds(s * NCPT, NCPT)
        e_copies = [
            pltpu.make_async_copy(col_hbm.at[sl_e], col_v, sem_e),
            pltpu.make_async_copy(row_hbm.at[sl_e], row_v, sem_e),
            pltpu.make_async_copy(val_hbm.at[sl_e], val_v, sem_e),
        ]
        for cp in e_copies:
            cp.start()

        zero16 = jnp.zeros((L,), jnp.float32)

        def zrow(i, carry):
            for g in range(H // L):
                rows0_v[i, pl.ds(g * L, L)] = zero16
            return carry

        lax.fori_loop(0, C, zrow, 0)
        for q in range(RPT // C):
            pltpu.sync_copy(rows0_v, acc_sh.at[pl.ds(s * RPT + q * C, C)])
        rem = RPT % C
        if rem:
            pltpu.sync_copy(
                rows0_v.at[pl.ds(0, rem)],
                acc_sh.at[pl.ds(s * RPT + (RPT // C) * C, rem)])
        plsc.subcore_barrier()
        for cp in e_copies:
            cp.wait()

        def start_gather(i, b):
            @pl.when(c == 0)
            def _():
                pltpu.make_async_copy(
                    xlo_hbm.at[col_v.at[i]], rows_bufs[b], sems[b]).start()

            @pl.when(c == 1)
            def _():
                pltpu.make_async_copy(
                    xhi_hbm.at[col_v.at[i]], rows_bufs[b], sems[b]).start()

        def wait_gather(i, b):
            pltpu.make_async_copy(
                xlo_hbm.at[col_v.at[i]], rows_bufs[b], sems[b]).wait()

        def process(i, b):
            rows_b = rows_bufs[b]
            wait_gather(i, b)

            @plsc.parallel_loop(0, C // L, unroll=C // L)
            def scale(j):
                v16 = val_v[i, pl.ds(j * L, L)]
                for el in range(L):
                    v = v16[el]
                    e = j * L + el
                    loads = [rows_b[e, pl.ds(g * L, L)]
                             for g in range(H // L)]
                    prods = [x * v for x in loads]
                    for g in range(H // L):
                        rows_b[e, pl.ds(g * L, L)] = prods[g]
            pltpu.sync_copy(rows_b, acc_sh.at[row_v.at[i]], add=True)

        start_gather(0, 0)

        def pair(kk, carry):
            i0 = 2 * kk
            start_gather(i0 + 1, 1)
            process(i0, 0)

            @pl.when(i0 + 2 < NCPT)
            def _():
                start_gather(i0 + 2, 0)

            process(i0 + 1, 1)
            return carry

        lax.fori_loop(0, NCPT // 2, pair, 0)
        plsc.subcore_barrier()

        lo = s * RPT
        pltpu.sync_copy(acc_sh.at[pl.ds(lo, RPT)],
                        out_hbm.at[pl.ds(lo, RPT), pl.ds(c * H, H)])

    return k(x_lo, x_hi, row2, col2, val2)


def kernel(seq, adj_row, adj_col, adj_val):
    x = jnp.squeeze(seq, 0)
    out = _sc_spmm(x[:, :H], x[:, H:],
                   adj_row.reshape(-1, C), adj_col.reshape(-1, C),
                   adj_val.reshape(-1, C))
    return jnp.expand_dims(out, 0)


# speedup vs baseline: 12.1387x; 1.2111x over previous
import functools

import jax
import jax.numpy as jnp
from jax import lax
from jax.experimental import pallas as pl
from jax.experimental.pallas import tpu as pltpu
from jax.experimental.pallas import tpu_sc as plsc

N = 10000
D = 128
E = 320000

NC = 2
NS = 16
L = 16
H = D // NC
EPT = E // NS
C = 80
NCPT = EPT // C
RPT = N // NS


def _sc_spmm(x_lo, x_hi, row2, col2, val2):
    mesh = plsc.VectorSubcoreMesh(core_axis_name="c", subcore_axis_name="s")

    @functools.partial(
        pl.kernel,
        mesh=mesh,
        out_type=jax.ShapeDtypeStruct((N, D), jnp.float32),
        compiler_params=pltpu.CompilerParams(use_tc_tiling_on_sc=False),
        scratch_types=[
            pltpu.VMEM((NCPT, C), jnp.int32),
            pltpu.VMEM((NCPT, C), jnp.int32),
            pltpu.VMEM((NCPT, C), jnp.float32),
            pltpu.VMEM((C, H), jnp.float32),
            pltpu.VMEM((C, H), jnp.float32),
            pltpu.VMEM((C, H), jnp.float32),
            pltpu.VMEM_SHARED((N, H), jnp.float32),
            pltpu.SemaphoreType.DMA,
            pltpu.SemaphoreType.DMA,
            pltpu.SemaphoreType.DMA,
            pltpu.SemaphoreType.DMA,
            pltpu.SemaphoreType.DMA,
            pltpu.SemaphoreType.DMA,
            pltpu.SemaphoreType.DMA,
        ],
    )
    def k(xlo_hbm, xhi_hbm, row_hbm, col_hbm, val_hbm, out_hbm,
          col_v, row_v, val_v, rows0_v, rows1_v, rows2_v, acc_sh,
          sem_e, sem_g0, sem_g1, sem_g2, sem_a0, sem_a1, sem_a2):
        c = lax.axis_index("c")
        s = lax.axis_index("s")
        rows_bufs = (rows0_v, rows1_v, rows2_v)
        sems = (sem_g0, sem_g1, sem_g2)
        asems = (sem_a0, sem_a1, sem_a2)

        sl_e = pl.ds(s * NCPT, NCPT)
        e_copies = [
            pltpu.make_async_copy(col_hbm.at[sl_e], col_v, sem_e),
            pltpu.make_async_copy(row_hbm.at[sl_e], row_v, sem_e),
            pltpu.make_async_copy(val_hbm.at[sl_e], val_v, sem_e),
        ]
        for cp in e_copies:
            cp.start()

        zero16 = jnp.zeros((L,), jnp.float32)

        def zrow(i, carry):
            for g in range(H // L):
                rows0_v[i, pl.ds(g * L, L)] = zero16
            return carry

        lax.fori_loop(0, C, zrow, 0)
        for q in range(RPT // C):
            pltpu.sync_copy(rows0_v, acc_sh.at[pl.ds(s * RPT + q * C, C)])
        rem = RPT % C
        if rem:
            pltpu.sync_copy(
                rows0_v.at[pl.ds(0, rem)],
                acc_sh.at[pl.ds(s * RPT + (RPT // C) * C, rem)])
        plsc.subcore_barrier()
        for cp in e_copies:
            cp.wait()

        def start_gather(i, b):
            @pl.when(c == 0)
            def _():
                pltpu.make_async_copy(
                    xlo_hbm.at[col_v.at[i]], rows_bufs[b], sems[b]).start()

            @pl.when(c == 1)
            def _():
                pltpu.make_async_copy(
                    xhi_hbm.at[col_v.at[i]], rows_bufs[b], sems[b]).start()

        def wait_gather(i, b):
            pltpu.make_async_copy(
                xlo_hbm.at[col_v.at[i]], rows_bufs[b], sems[b]).wait()

        def start_scatter(i, b):
            pltpu.async_copy(
                rows_bufs[b], acc_sh.at[row_v.at[i]], asems[b], add=True)

        def wait_scatter(i, b):
            pltpu.make_async_copy(
                rows_bufs[b], acc_sh.at[row_v.at[i]], asems[b]).wait()

        def scale_chunk(i, b):
            rows_b = rows_bufs[b]

            @plsc.parallel_loop(0, C // L, unroll=C // L)
            def scale(j):
                v16 = val_v[i, pl.ds(j * L, L)]
                for el in range(L):
                    v = v16[el]
                    e = j * L + el
                    loads = [rows_b[e, pl.ds(g * L, L)]
                             for g in range(H // L)]
                    prods = [x * v for x in loads]
                    for g in range(H // L):
                        rows_b[e, pl.ds(g * L, L)] = prods[g]

        def block(i, b):
            bn = (b + 1) % 3

            @pl.when(i >= 2)
            def _():
                wait_scatter(jnp.maximum(i - 2, 0), bn)

            start_gather(i + 1, bn)
            wait_gather(i, b)
            scale_chunk(i, b)
            start_scatter(i, b)

        start_gather(0, 0)

        def triple(kk, carry):
            i0 = 3 * kk
            block(i0, 0)
            block(i0 + 1, 1)
            block(i0 + 2, 2)
            return carry

        lax.fori_loop(0, (NCPT - 1) // 3, triple, 0)
        ilast = NCPT - 1
        wait_scatter(ilast - 2, (ilast - 2) % 3)
        wait_gather(ilast, ilast % 3)
        scale_chunk(ilast, ilast % 3)
        start_scatter(ilast, ilast % 3)
        wait_scatter(ilast - 1, (ilast - 1) % 3)
        wait_scatter(ilast, ilast % 3)
        plsc.subcore_barrier()

        lo = s * RPT
        pltpu.sync_copy(acc_sh.at[pl.ds(lo, RPT)],
                        out_hbm.at[pl.ds(lo, RPT), pl.ds(c * H, H)])

    return k(x_lo, x_hi, row2, col2, val2)


def kernel(seq, adj_row, adj_col, adj_val):
    x = jnp.squeeze(seq, 0)
    out = _sc_spmm(x[:, :H], x[:, H:],
                   adj_row.reshape(-1, C), adj_col.reshape(-1, C),
                   adj_val.reshape(-1, C))
    return jnp.expand_dims(out, 0)
